# baseline (speedup 1.0000x reference)
import functools

import jax
import jax.numpy as jnp
from jax import lax
from jax.experimental import pallas as pl
from jax.experimental.pallas import tpu as pltpu
from jax.experimental.pallas import tpu_sc as plsc

B, N, L = 1024, 26, 20
VOCAB, D = 128, 128
TOT = B * N * L
NC, NS = 2, 16
NW = NC * NS
PER_W = TOT // NW
CHUNK = 128
NCHUNK = PER_W // CHUNK
NBUF = 5
GROUPS = NCHUNK // NBUF
COMPUTE_BUFS = (1, 3)
LANES = 16

_mesh = plsc.VectorSubcoreMesh(core_axis_name="c", subcore_axis_name="s")


@functools.partial(
    pl.kernel,
    mesh=_mesh,
    out_type=jax.ShapeDtypeStruct((TOT, D), jnp.float32),
    scratch_types=(
        [pltpu.VMEM((NCHUNK, CHUNK), jnp.int32)]
        + [pltpu.VMEM((CHUNK, D), jnp.float32) for _ in range(NBUF)]
        + [pltpu.SemaphoreType.DMA for _ in range(2 * NBUF)]
        + [pltpu.VMEM_SHARED((VOCAB, D), jnp.float32)]
        + [pltpu.VMEM((VOCAB, D), jnp.float32)]
    ),
    compiler_params=pltpu.CompilerParams(needs_layout_passes=False),
)
def _embed(table_hbm, idx_hbm, out_hbm, idx_v, *rest):
    rows = rest[:NBUF]
    sem_g = rest[NBUF:2 * NBUF]
    sem_s = rest[2 * NBUF:3 * NBUF]
    table_sh = rest[3 * NBUF]
    table_v = rest[3 * NBUF + 1]
    wid = lax.axis_index("s") * NC + lax.axis_index("c")
    base = wid * PER_W

    @pl.when(lax.axis_index("s") == 0)
    def _stage():
        pltpu.sync_copy(table_hbm, table_sh)

    pltpu.sync_copy(table_hbm, table_v)
    plsc.subcore_barrier()

    def gather(chunk, b):
        pltpu.async_copy(table_sh.at[idx_v.at[chunk]], rows[b], sem_g[b])

    def gather_wait(b):
        pltpu.make_async_copy(table_sh.at[idx_v.at[0]], rows[b], sem_g[b]).wait()

    def store(chunk, b):
        dst = out_hbm.at[pl.ds(base + chunk * CHUNK, CHUNK)]
        pltpu.async_copy(rows[b], dst, sem_s[b])

    def store_wait(b):
        dst = out_hbm.at[pl.ds(base, CHUNK)]
        pltpu.make_async_copy(rows[b], dst, sem_s[b]).wait()

    def compute_gather(chunk, b):
        iota = lax.iota(jnp.int32, LANES)
        chv = lax.broadcast(chunk, (LANES,))

        def col_loop(c0, carry):
            cb = lax.broadcast(c0 * LANES, (LANES,))

            def row_loop(jb, carry2):
                rid = jb * LANES + iota
                idxv = plsc.load_gather(idx_v, [chv, rid])
                for sh in range(LANES):
                    col = ((iota + sh) & (LANES - 1)) + cb
                    vals = plsc.load_gather(table_v, [idxv, col])
                    plsc.store_scatter(rows[b], [rid, col], vals)
                return carry2

            lax.fori_loop(0, CHUNK // LANES, row_loop, 0)
            return carry

        lax.fori_loop(0, D // LANES, col_loop, 0)

    pltpu.sync_copy(idx_hbm.at[wid], idx_v)
    for b in range(NBUF):
        if b not in COMPUTE_BUFS:
            gather(b, b)

    def body(g, carry):
        for b in range(NBUF):
            j = g * NBUF + b
            bp = (b - 1) % NBUF
            if bp not in COMPUTE_BUFS:
                if b == 0:
                    can_refill = g >= 1
                else:
                    can_refill = g <= GROUPS - 2
                @pl.when(can_refill)
                def _refill(bp=bp, chunk=j + NBUF - 1):
                    store_wait(bp)
                    gather(chunk, bp)
            if b in COMPUTE_BUFS:
                @pl.when(g >= 1)
                def _w(b=b):
                    store_wait(b)
                compute_gather(j, b)
            else:
                gather_wait(b)
            store(j, b)
        return carry

    lax.fori_loop(0, GROUPS, body, 0)
    for b in range(NBUF):
        store_wait(b)


def kernel(input_feature, table):
    idx_t = jnp.transpose(input_feature, (1, 2, 0))
    idx = idx_t.reshape(NW, NCHUNK, CHUNK).astype(jnp.int32)
    out = _embed(table, idx)
    return out.reshape(N, L, B, D).transpose(2, 0, 1, 3)


# speedup vs baseline: 2.7110x; 2.7110x over previous
import functools

import jax
import jax.numpy as jnp
from jax import lax
from jax.experimental import pallas as pl
from jax.experimental.pallas import tpu as pltpu
from jax.experimental.pallas import tpu_sc as plsc

B, N, L = 1024, 26, 20
VOCAB, D = 128, 128
TOT = B * N * L
NC, NS = 2, 16
NW = NC * NS
PER_W = TOT // NW
CHUNK = 128
NCHUNK = PER_W // CHUNK
NBUF = 5
GROUPS = NCHUNK // NBUF

_mesh = plsc.VectorSubcoreMesh(core_axis_name="c", subcore_axis_name="s")


@functools.partial(
    pl.kernel,
    mesh=_mesh,
    out_type=jax.ShapeDtypeStruct((TOT, D), jnp.float32),
    scratch_types=(
        [pltpu.VMEM((NCHUNK, CHUNK), jnp.int32)]
        + [pltpu.VMEM((CHUNK, D), jnp.float32) for _ in range(NBUF)]
        + [pltpu.SemaphoreType.DMA for _ in range(2 * NBUF)]
        + [pltpu.VMEM_SHARED((VOCAB, D), jnp.float32)]
    ),
)
def _embed(table_hbm, idx_hbm, out_hbm, idx_v, *bufs_and_sems):
    rows = bufs_and_sems[:NBUF]
    sem_g = bufs_and_sems[NBUF:2 * NBUF]
    sem_s = bufs_and_sems[2 * NBUF:3 * NBUF]
    table_sh = bufs_and_sems[3 * NBUF]
    wid = lax.axis_index("s") * NC + lax.axis_index("c")
    base = wid * PER_W

    @pl.when(lax.axis_index("s") == 0)
    def _stage():
        pltpu.sync_copy(table_hbm, rows[0])
        pltpu.sync_copy(rows[0], table_sh)

    plsc.subcore_barrier()

    def gather(chunk, b):
        pltpu.async_copy(table_sh.at[idx_v.at[chunk]], rows[b], sem_g[b])

    def gather_wait(b):
        pltpu.make_async_copy(table_sh.at[idx_v.at[0]], rows[b], sem_g[b]).wait()

    def store(chunk, b):
        dst = out_hbm.at[pl.ds(base + chunk * CHUNK, CHUNK)]
        pltpu.async_copy(rows[b], dst, sem_s[b])

    def store_wait(b):
        dst = out_hbm.at[pl.ds(base, CHUNK)]
        pltpu.make_async_copy(rows[b], dst, sem_s[b]).wait()

    pltpu.sync_copy(idx_hbm.at[wid], idx_v)
    for b in range(NBUF):
        gather(b, b)

    def body(g, carry):
        for b in range(NBUF):
            j = g * NBUF + b
            bp = (b - 1) % NBUF
            if b == 0:
                can_refill = g >= 1
            else:
                can_refill = g <= GROUPS - 2
            @pl.when(can_refill)
            def _refill(bp=bp, chunk=j + NBUF - 1):
                store_wait(bp)
                gather(chunk, bp)
            gather_wait(b)
            store(j, b)
        return carry

    lax.fori_loop(0, GROUPS, body, 0)
    for b in range(NBUF):
        store_wait(b)


def kernel(input_feature, table):
    idx_t = jnp.transpose(input_feature, (1, 2, 0))
    idx = idx_t.reshape(NW, NCHUNK, CHUNK).astype(jnp.int32)
    out = _embed(table, idx)
    return out.reshape(N, L, B, D).transpose(2, 0, 1, 3)


# speedup vs baseline: 2.7183x; 1.0027x over previous
import functools

import jax
import jax.numpy as jnp
from jax import lax
from jax.experimental import pallas as pl
from jax.experimental.pallas import tpu as pltpu
from jax.experimental.pallas import tpu_sc as plsc

B, N, L = 1024, 26, 20
VOCAB, D = 128, 128
TOT = B * N * L
NC, NS = 2, 16
NW = NC * NS
PER_W = TOT // NW
CHUNK = 128
NCHUNK = PER_W // CHUNK
BIGC = 2 * CHUNK
NBIG = PER_W // BIGC
NBUF = 3
GROUPS = (NBIG - 2) // NBUF

_mesh = plsc.VectorSubcoreMesh(core_axis_name="c", subcore_axis_name="s")


@functools.partial(
    pl.kernel,
    mesh=_mesh,
    out_type=jax.ShapeDtypeStruct((TOT, D), jnp.float32),
    scratch_types=(
        [pltpu.VMEM((NCHUNK, CHUNK), jnp.int32)]
        + [pltpu.VMEM((BIGC, D), jnp.float32) for _ in range(NBUF)]
        + [pltpu.SemaphoreType.DMA for _ in range(2 * NBUF)]
        + [pltpu.VMEM_SHARED((VOCAB, D), jnp.float32)]
    ),
)
def _embed(table_hbm, idx_hbm, out_hbm, idx_v, *bufs_and_sems):
    rows = bufs_and_sems[:NBUF]
    sem_g = bufs_and_sems[NBUF:2 * NBUF]
    sem_s = bufs_and_sems[2 * NBUF:3 * NBUF]
    table_sh = bufs_and_sems[3 * NBUF]
    wid = lax.axis_index("s") * NC + lax.axis_index("c")
    base = wid * PER_W

    @pl.when(lax.axis_index("s") == 0)
    def _stage():
        pltpu.sync_copy(table_hbm, table_sh)

    plsc.subcore_barrier()

    def gather(k, b):
        pltpu.async_copy(table_sh.at[idx_v.at[2 * k]],
                         rows[b].at[pl.ds(0, CHUNK)], sem_g[b])
        pltpu.async_copy(table_sh.at[idx_v.at[2 * k + 1]],
                         rows[b].at[pl.ds(CHUNK, CHUNK)], sem_g[b])

    def gather_wait(b):
        for h in range(2):
            pltpu.make_async_copy(table_sh.at[idx_v.at[0]],
                                  rows[b].at[pl.ds(h * CHUNK, CHUNK)],
                                  sem_g[b]).wait()

    def store(k, b):
        dst = out_hbm.at[pl.ds(base + k * BIGC, BIGC)]
        pltpu.async_copy(rows[b], dst, sem_s[b])

    def store_wait(b):
        dst = out_hbm.at[pl.ds(base, BIGC)]
        pltpu.make_async_copy(rows[b], dst, sem_s[b]).wait()

    pltpu.sync_copy(idx_hbm.at[wid], idx_v)
    for b in range(NBUF):
        gather(b, b)

    def body(g, carry):
        for b in range(NBUF):
            j = g * NBUF + b
            bp = (b - 1) % NBUF
            if b == 0:
                can_refill = jnp.logical_and(g >= 1, j <= NBIG - 3)
            else:
                can_refill = j <= NBIG - 3
            @pl.when(can_refill)
            def _refill(bp=bp, k=j + NBUF - 1):
                store_wait(bp)
                gather(k, bp)
            gather_wait(b)
            store(j, b)
        return carry

    lax.fori_loop(0, GROUPS, body, 0)
    for j in range(GROUPS * NBUF, NBIG):
        b = j % NBUF
        gather_wait(b)
        store(j, b)
    for b in range(NBUF):
        store_wait(b)


def kernel(input_feature, table):
    idx_t = jnp.transpose(input_feature, (1, 2, 0))
    idx = idx_t.reshape(NW, NCHUNK, CHUNK).astype(jnp.int32)
    out = _embed(table, idx)
    return out.reshape(N, L, B, D).transpose(2, 0, 1, 3)
